# baseline (speedup 1.0000x reference)
import jax
import jax.numpy as jnp
from jax.experimental import pallas as pl
from jax.experimental.pallas import tpu as pltpu

_R_BLK = 3336
_K_BLK = 1024


def _support_body(x_ref, w_ref, sup_ref):
    n = x_ref.shape[0]
    in_f = w_ref.shape[0] // 2
    x = x_ref[...]
    w = w_ref[...]
    s0 = jax.lax.dot(x[:, :in_f], w[:in_f, :],
                     precision=jax.lax.Precision.DEFAULT,
                     preferred_element_type=jnp.float32)
    s1 = jax.lax.dot(x[:, in_f:], w[in_f:, :],
                     precision=jax.lax.Precision.DEFAULT,
                     preferred_element_type=jnp.float32)
    sup_ref[0:n, :] = jnp.concatenate([s0, s1], axis=1).astype(jnp.bfloat16)
    if sup_ref.shape[0] > n:
        sup_ref[n:, :] = jnp.zeros(
            (sup_ref.shape[0] - n, sup_ref.shape[1]), jnp.bfloat16)


def _make_spmm_body(n):
    def _spmm_body(sup_ref, adj_ref, out_ref):
        a = adj_ref[...].astype(jnp.bfloat16)
        lim = n - pl.program_id(1) * _K_BLK
        col = jax.lax.broadcasted_iota(jnp.int32, a.shape, 1)
        a = jnp.where(col < lim, a, jnp.bfloat16(0))
        part = jax.lax.dot(a, sup_ref[...], preferred_element_type=jnp.float32)

        @pl.when(pl.program_id(1) == 0)
        def _():
            out_ref[...] = part

        @pl.when(pl.program_id(1) > 0)
        def _():
            out_ref[...] += part

    return _spmm_body


def kernel(input, adj, weight):
    n, batch, in_f = input.shape
    out_f = weight.shape[-1]
    assert batch == 2
    bf = batch * out_f

    num_r = -(-n // _R_BLK)
    num_k = -(-n // _K_BLK)
    k_pad = num_k * _K_BLK

    x2d = input.reshape(n, batch * in_f)
    w2d = weight.reshape(batch * in_f, out_f)

    sup = pl.pallas_call(
        _support_body,
        out_shape=jax.ShapeDtypeStruct((k_pad, bf), jnp.bfloat16),
    )(x2d, w2d)

    out = pl.pallas_call(
        _make_spmm_body(n),
        grid=(num_r, num_k),
        in_specs=[
            pl.BlockSpec((_K_BLK, bf), lambda r, k: (k, 0)),
            pl.BlockSpec((_R_BLK, _K_BLK), lambda r, k: (r, k)),
        ],
        out_specs=pl.BlockSpec((_R_BLK, bf), lambda r, k: (r, 0)),
        out_shape=jax.ShapeDtypeStruct((n, bf), jnp.float32),
        compiler_params=pltpu.CompilerParams(
            dimension_semantics=("arbitrary", "arbitrary"),
        ),
    )(sup, adj)

    return out.reshape(n, batch, out_f)


# speedup vs baseline: 1.0027x; 1.0027x over previous
import jax
import jax.numpy as jnp
from jax.experimental import pallas as pl
from jax.experimental.pallas import tpu as pltpu

_R_BLK = 3336
_K_BLK = 1024


def _support_body(x_ref, w_ref, sup_ref):
    n = x_ref.shape[0]
    in_f = w_ref.shape[0] // 2
    x = x_ref[...]
    w = w_ref[...]
    s0 = jax.lax.dot(x[:, :in_f], w[:in_f, :],
                     precision=jax.lax.Precision.DEFAULT,
                     preferred_element_type=jnp.float32)
    s1 = jax.lax.dot(x[:, in_f:], w[in_f:, :],
                     precision=jax.lax.Precision.DEFAULT,
                     preferred_element_type=jnp.float32)
    sup_ref[0:n, :] = jnp.concatenate([s0, s1], axis=1).astype(jnp.bfloat16)
    if sup_ref.shape[0] > n:
        sup_ref[n:, :] = jnp.zeros(
            (sup_ref.shape[0] - n, sup_ref.shape[1]), jnp.bfloat16)


def _make_spmm_body(n):
    def _spmm_body(sup_ref, adj_ref, out_ref):
        a = adj_ref[...]
        lim = n - pl.program_id(1) * _K_BLK
        col = jax.lax.broadcasted_iota(jnp.int32, a.shape, 1)
        a = jnp.where(col < lim, a, jnp.float32(0))
        part = jax.lax.dot(a, sup_ref[...],
                           precision=jax.lax.Precision.DEFAULT,
                           preferred_element_type=jnp.float32)

        @pl.when(pl.program_id(1) == 0)
        def _():
            out_ref[...] = part

        @pl.when(pl.program_id(1) > 0)
        def _():
            out_ref[...] += part

    return _spmm_body


def kernel(input, adj, weight):
    n, batch, in_f = input.shape
    out_f = weight.shape[-1]
    assert batch == 2
    bf = batch * out_f

    num_r = -(-n // _R_BLK)
    num_k = -(-n // _K_BLK)
    k_pad = num_k * _K_BLK

    x2d = input.reshape(n, batch * in_f)
    w2d = weight.reshape(batch * in_f, out_f)

    sup = pl.pallas_call(
        _support_body,
        out_shape=jax.ShapeDtypeStruct((k_pad, bf), jnp.bfloat16),
    )(x2d, w2d)

    out = pl.pallas_call(
        _make_spmm_body(n),
        grid=(num_r, num_k),
        in_specs=[
            pl.BlockSpec((_K_BLK, bf), lambda r, k: (k, 0)),
            pl.BlockSpec((_R_BLK, _K_BLK), lambda r, k: (r, k)),
        ],
        out_specs=pl.BlockSpec((_R_BLK, bf), lambda r, k: (r, 0)),
        out_shape=jax.ShapeDtypeStruct((n, bf), jnp.float32),
        compiler_params=pltpu.CompilerParams(
            dimension_semantics=("arbitrary", "arbitrary"),
        ),
    )(sup, adj)

    return out.reshape(n, batch, out_f)


# speedup vs baseline: 1.0187x; 1.0159x over previous
import jax
import jax.numpy as jnp
from jax.experimental import pallas as pl
from jax.experimental.pallas import tpu as pltpu

_R_BLK = 200


def _support_body(x_ref, w_ref, sup_ref):
    in_f = w_ref.shape[0] // 2
    x = x_ref[...]
    w = w_ref[...]
    s0 = jax.lax.dot(x[:, :in_f], w[:in_f, :],
                     precision=jax.lax.Precision.DEFAULT,
                     preferred_element_type=jnp.float32)
    s1 = jax.lax.dot(x[:, in_f:], w[in_f:, :],
                     precision=jax.lax.Precision.DEFAULT,
                     preferred_element_type=jnp.float32)
    sup_ref[...] = jnp.concatenate([s0, s1], axis=1).astype(jnp.bfloat16)


def _spmm_body(sup_ref, adj_top_ref, adj_hbm, out_ref, ring_ref, sems):
    n = adj_top_ref.shape[1]
    half = n // 2
    nsteps = half // _R_BLK
    r = pl.program_id(0)

    def ring_copy(step, buf):
        return pltpu.make_async_copy(
            adj_hbm.at[pl.ds(half + step * _R_BLK, _R_BLK), :],
            ring_ref.at[buf],
            sems.at[buf],
        )

    @pl.when(r == 0)
    def _():
        ring_copy(0, 0).start(priority=1)
        ring_copy(1, 1).start(priority=1)

    sup = sup_ref[...]
    out_ref[0] = jax.lax.dot(adj_top_ref[...].astype(jnp.bfloat16), sup,
                             preferred_element_type=jnp.float32)

    buf = jax.lax.rem(r, 2)
    ring_copy(r, buf).wait()
    out_ref[1] = jax.lax.dot(ring_ref[buf].astype(jnp.bfloat16), sup,
                             preferred_element_type=jnp.float32)

    @pl.when(r + 2 < nsteps)
    def _():
        ring_copy(r + 2, buf).start(priority=1)


def kernel(input, adj, weight):
    n, batch, in_f = input.shape
    out_f = weight.shape[-1]
    assert batch == 2
    bf = batch * out_f
    half = n // 2

    x2d = input.reshape(n, batch * in_f)
    w2d = weight.reshape(batch * in_f, out_f)

    sup = pl.pallas_call(
        _support_body,
        out_shape=jax.ShapeDtypeStruct((n, bf), jnp.bfloat16),
    )(x2d, w2d)

    out = pl.pallas_call(
        _spmm_body,
        grid=(half // _R_BLK,),
        in_specs=[
            pl.BlockSpec((n, bf), lambda r: (0, 0)),
            pl.BlockSpec((_R_BLK, n), lambda r: (r, 0)),
            pl.BlockSpec(memory_space=pl.ANY),
        ],
        out_specs=pl.BlockSpec((2, _R_BLK, bf), lambda r: (0, r, 0)),
        out_shape=jax.ShapeDtypeStruct((2, half, bf), jnp.float32),
        scratch_shapes=[
            pltpu.VMEM((2, _R_BLK, n), jnp.float32),
            pltpu.SemaphoreType.DMA((2,)),
        ],
        compiler_params=pltpu.CompilerParams(
            dimension_semantics=("arbitrary",),
        ),
    )(sup, adj, adj)

    return out.reshape(n, batch, out_f)
